# baseline (speedup 1.0000x reference)
import functools

import jax
import jax.numpy as jnp
from jax import lax
from jax.experimental import pallas as pl
from jax.experimental.pallas import tpu as pltpu
from jax.experimental.pallas import tpu_sc as plsc

N_FEATURES = 26
VOCAB = 100000
OUT_DIM = 64
BATCH = 16384

LANES = 16
D_PER_W = 4
F_GROUPS = 2
F_PER_W = N_FEATURES // F_GROUPS
D_GROUPS = OUT_DIM // D_PER_W
OCHUNK = 4096
N_OCH = BATCH // OCHUNK
GRP = 16

assert F_GROUPS * D_GROUPS == 32
assert F_PER_W * F_GROUPS == N_FEATURES
assert D_PER_W * D_GROUPS == OUT_DIM
assert N_OCH * OCHUNK == BATCH


def _body(idx_hbm, tab_hbm, out_hbm, idx_v, row_v, ob0, ob1, ws0, ws1, rsem):
    c = lax.axis_index("c")
    s = lax.axis_index("s")
    wid = s * 2 + c
    d0 = (wid % D_GROUPS) * D_PER_W
    f0 = (wid // D_GROUPS) * F_PER_W
    obufs = (ob0, ob1)
    wsems = (ws0, ws1)

    def per_feature(f, first):
        pltpu.sync_copy(idx_hbm.at[f], idx_v)
        for dd in range(D_PER_W):
            d = d0 + dd
            pltpu.async_copy(tab_hbm.at[f, d], row_v, rsem).wait()
            for ci in range(N_OCH):
                ob = obufs[ci % 2]
                wsem = wsems[ci % 2]
                dst = out_hbm.at[f, d, pl.ds(ci * OCHUNK, OCHUNK)]
                if not (first and dd == 0 and ci < 2):
                    pltpu.make_async_copy(ob, dst, wsem).wait()

                @plsc.parallel_loop(0, OCHUNK // LANES, 1, unroll=GRP)
                def gather_grp(g, ci=ci, ob=ob):
                    off = idx_v[pl.ds(ci * OCHUNK + g * LANES, LANES)]
                    ob[pl.ds(g * LANES, LANES)] = plsc.load_gather(row_v, [off])

                pltpu.async_copy(ob, dst, wsem)

    per_feature(f0, True)

    def rest(ff, carry):
        per_feature(f0 + ff, False)
        return carry

    lax.fori_loop(1, F_PER_W, rest, 0)

    for ci in range(2):
        r = N_OCH - 2 + ci
        pltpu.make_async_copy(
            obufs[r % 2],
            out_hbm.at[f0 + F_PER_W - 1, d0 + D_PER_W - 1, pl.ds(r * OCHUNK, OCHUNK)],
            wsems[r % 2],
        ).wait()


@jax.jit
def _embed(idx_t, tab_t):
    run = functools.partial(
        pl.kernel,
        mesh=plsc.VectorSubcoreMesh(core_axis_name="c", subcore_axis_name="s"),
        out_type=jax.ShapeDtypeStruct((N_FEATURES, OUT_DIM, BATCH), jnp.float32),
        scratch_types=[
            pltpu.VMEM((BATCH,), jnp.int32),
            pltpu.VMEM((VOCAB,), jnp.float32),
            pltpu.VMEM((OCHUNK,), jnp.float32),
            pltpu.VMEM((OCHUNK,), jnp.float32),
            pltpu.SemaphoreType.DMA,
            pltpu.SemaphoreType.DMA,
            pltpu.SemaphoreType.DMA,
        ],
        compiler_params=pltpu.CompilerParams(needs_layout_passes=False),
    )(_body)
    return run(idx_t, tab_t)


def kernel(inp, tables):
    idx_t = inp.astype(jnp.int32).T
    tab_t = jnp.swapaxes(tables, 1, 2)
    out_t = _embed(idx_t, tab_t)
    return jnp.transpose(out_t, (2, 0, 1))
